# baseline (speedup 1.0000x reference)
import dataclasses
import functools

import jax
import jax.numpy as jnp
from jax import lax
from jax.experimental import pallas as pl
from jax.experimental.pallas import tpu as pltpu
from jax.experimental.pallas import tpu_sc as plsc

NUM_CORES = 2
NUM_SUBCORES = 16
NUM_TILES = NUM_CORES * NUM_SUBCORES
LANES = 16
D = 2 * LANES
PACK = 128 // D
CHUNK = 128


@functools.lru_cache(maxsize=None)
def _build(batch: int):
    assert batch % (8 * NUM_TILES) == 0
    bpw = batch // NUM_TILES
    n_chunks = bpw // CHUNK

    mesh = plsc.VectorSubcoreMesh(core_axis_name="c", subcore_axis_name="s")
    cp = pltpu.CompilerParams()
    if "needs_layout_passes" in pltpu.CompilerParams.__dataclass_fields__:
        cp = dataclasses.replace(cp, needs_layout_passes=False)

    @functools.partial(
        pl.kernel,
        out_type=jax.ShapeDtypeStruct((batch,), jnp.float32),
        mesh=mesh,
        compiler_params=cp,
        scratch_types=[
            pltpu.VMEM((bpw,), jnp.int32),
            pltpu.VMEM((bpw,), jnp.int32),
            pltpu.VMEM((bpw,), jnp.int32),
            pltpu.VMEM((bpw,), jnp.int32),
            pltpu.VMEM((CHUNK, 128), jnp.float32),
            pltpu.VMEM((CHUNK, 128), jnp.float32),
            pltpu.VMEM((bpw,), jnp.float32),
            pltpu.VMEM((128,), jnp.float32),
            pltpu.SemaphoreType.DMA,
            pltpu.SemaphoreType.DMA,
        ],
    )
    def dlcrs(uq_h, mq_h, uo_h, mo_h, ut_h, mt_h, wb_h, out_h,
              uqv, mqv, uov, mov, urows, mrows, outv, wbv,
              sem_u, sem_m):
        wid = lax.axis_index("s") * NUM_CORES + lax.axis_index("c")
        base = wid * bpw

        pltpu.sync_copy(uq_h.at[pl.ds(base, bpw)], uqv)
        pltpu.sync_copy(mq_h.at[pl.ds(base, bpw)], mqv)
        pltpu.sync_copy(uo_h.at[pl.ds(base, bpw)], uov)
        pltpu.sync_copy(mo_h.at[pl.ds(base, bpw)], mov)
        pltpu.sync_copy(wb_h, wbv)

        wu0 = wbv[pl.ds(0, LANES)]
        wu1 = wbv[pl.ds(LANES, LANES)]
        wm0 = wbv[pl.ds(2 * LANES, LANES)]
        wm1 = wbv[pl.ds(3 * LANES, LANES)]
        bvv = wbv[pl.ds(4 * LANES, LANES)]
        iota = lax.iota(jnp.int32, LANES)
        last = iota == (LANES - 1)

        for c in range(n_chunks):
            sl = pl.ds(c * CHUNK, CHUNK)
            cu = pltpu.async_copy(ut_h.at[uqv.at[sl]], urows, sem_u)
            cm = pltpu.async_copy(mt_h.at[mqv.at[sl]], mrows, sem_m)
            cu.wait()
            cm.wait()

            @pl.loop(0, CHUNK)
            def _(r):
                g = c * CHUNK + r
                gfull = jnp.full((LANES,), g, jnp.int32)
                rfull = jnp.full((LANES,), r, jnp.int32)
                offu = plsc.load_gather(uov, [gfull]) + iota
                offm = plsc.load_gather(mov, [gfull]) + iota
                u0 = plsc.load_gather(urows, [rfull, offu])
                u1 = plsc.load_gather(urows, [rfull, offu + LANES])
                m0 = plsc.load_gather(mrows, [rfull, offm])
                m1 = plsc.load_gather(mrows, [rfull, offm + LANES])
                p = u0 * wu0 + u1 * wu1 + m0 * wm0 + m1 * wm1 + bvv
                s = jnp.cumsum(p)
                plsc.store_scatter(outv, [gfull], s, mask=last)

        pltpu.sync_copy(outv, out_h.at[pl.ds(base, bpw)])

    return dlcrs


def kernel(users, movies, user_table, movie_table, W, b):
    batch = users.shape[0]
    users = users.astype(jnp.int32)
    movies = movies.astype(jnp.int32)
    uq = users // PACK
    mq = movies // PACK
    uo = (users % PACK) * D
    mo = (movies % PACK) * D
    ut = user_table.reshape(user_table.shape[0] // PACK, 128)
    mt = movie_table.reshape(movie_table.shape[0] // PACK, 128)
    wb = jnp.concatenate([
        W.reshape(2 * D).astype(jnp.float32),
        jnp.broadcast_to(b / LANES, (LANES,)).astype(jnp.float32),
        jnp.zeros((128 - 2 * D - LANES,), jnp.float32),
    ])
    fn = _build(batch)
    out = fn(uq, mq, uo, mo, ut, mt, wb)
    return out.reshape(batch, 1)


# speedup vs baseline: 1.0019x; 1.0019x over previous
import dataclasses
import functools

import jax
import jax.numpy as jnp
from jax import lax
from jax.experimental import pallas as pl
from jax.experimental.pallas import tpu as pltpu
from jax.experimental.pallas import tpu_sc as plsc

NUM_CORES = 2
NUM_SUBCORES = 16
NUM_TILES = NUM_CORES * NUM_SUBCORES
LANES = 16
D = 2 * LANES
PACK = 128 // D
CHUNK = 128


@functools.lru_cache(maxsize=None)
def _build(batch: int):
    assert batch % (8 * NUM_TILES) == 0
    bpw = batch // NUM_TILES
    n_chunks = bpw // CHUNK

    mesh = plsc.VectorSubcoreMesh(core_axis_name="c", subcore_axis_name="s")
    cp = pltpu.CompilerParams()
    if "needs_layout_passes" in pltpu.CompilerParams.__dataclass_fields__:
        cp = dataclasses.replace(cp, needs_layout_passes=False)
    if "use_tc_tiling_on_sc" in pltpu.CompilerParams.__dataclass_fields__:
        cp = dataclasses.replace(cp, use_tc_tiling_on_sc=True)

    @functools.partial(
        pl.kernel,
        out_type=jax.ShapeDtypeStruct((batch,), jnp.float32),
        mesh=mesh,
        compiler_params=cp,
        scratch_types=[
            pltpu.VMEM((bpw,), jnp.int32),
            pltpu.VMEM((bpw,), jnp.int32),
            pltpu.VMEM((bpw,), jnp.int32),
            pltpu.VMEM((bpw,), jnp.int32),
            pltpu.VMEM((CHUNK, 128), jnp.float32),
            pltpu.VMEM((CHUNK, 128), jnp.float32),
            pltpu.VMEM((bpw,), jnp.float32),
            pltpu.VMEM((128,), jnp.float32),
            pltpu.SemaphoreType.DMA,
            pltpu.SemaphoreType.DMA,
        ],
    )
    def dlcrs(uq_h, mq_h, uo_h, mo_h, ut_h, mt_h, wb_h, out_h,
              uqv, mqv, uov, mov, urows, mrows, outv, wbv,
              sem_u, sem_m):
        wid = lax.axis_index("s") * NUM_CORES + lax.axis_index("c")
        base = wid * bpw

        pltpu.sync_copy(uq_h.at[pl.ds(base, bpw)], uqv)
        pltpu.sync_copy(mq_h.at[pl.ds(base, bpw)], mqv)
        pltpu.sync_copy(uo_h.at[pl.ds(base, bpw)], uov)
        pltpu.sync_copy(mo_h.at[pl.ds(base, bpw)], mov)
        pltpu.sync_copy(wb_h, wbv)

        wu0 = wbv[pl.ds(0, LANES)]
        wu1 = wbv[pl.ds(LANES, LANES)]
        wm0 = wbv[pl.ds(2 * LANES, LANES)]
        wm1 = wbv[pl.ds(3 * LANES, LANES)]
        bvv = wbv[pl.ds(4 * LANES, LANES)]
        iota = lax.iota(jnp.int32, LANES)
        last = iota == (LANES - 1)

        for c in range(n_chunks):
            sl = pl.ds(c * CHUNK, CHUNK)
            cu = pltpu.async_copy(ut_h.at[uqv.at[sl]], urows, sem_u)
            cm = pltpu.async_copy(mt_h.at[mqv.at[sl]], mrows, sem_m)
            cu.wait()
            cm.wait()

            @pl.loop(0, CHUNK)
            def _(r):
                g = c * CHUNK + r
                gfull = jnp.full((LANES,), g, jnp.int32)
                rfull = jnp.full((LANES,), r, jnp.int32)
                offu = plsc.load_gather(uov, [gfull]) + iota
                offm = plsc.load_gather(mov, [gfull]) + iota
                u0 = plsc.load_gather(urows, [rfull, offu])
                u1 = plsc.load_gather(urows, [rfull, offu + LANES])
                m0 = plsc.load_gather(mrows, [rfull, offm])
                m1 = plsc.load_gather(mrows, [rfull, offm + LANES])
                p = u0 * wu0 + u1 * wu1 + m0 * wm0 + m1 * wm1 + bvv
                s = jnp.cumsum(p)
                plsc.store_scatter(outv, [gfull], s, mask=last)

        pltpu.sync_copy(outv, out_h.at[pl.ds(base, bpw)])

    return dlcrs


def kernel(users, movies, user_table, movie_table, W, b):
    batch = users.shape[0]
    users = users.astype(jnp.int32)
    movies = movies.astype(jnp.int32)
    uq = users // PACK
    mq = movies // PACK
    uo = (users % PACK) * D
    mo = (movies % PACK) * D
    ut = user_table.reshape(user_table.shape[0] // PACK, 128)
    mt = movie_table.reshape(movie_table.shape[0] // PACK, 128)
    wb = jnp.concatenate([
        W.reshape(2 * D).astype(jnp.float32),
        jnp.broadcast_to(b / LANES, (LANES,)).astype(jnp.float32),
        jnp.zeros((128 - 2 * D - LANES,), jnp.float32),
    ])
    fn = _build(batch)
    out = fn(uq, mq, uo, mo, ut, mt, wb)
    return out.reshape(batch, 1)


# speedup vs baseline: 8.6448x; 8.6281x over previous
import dataclasses
import functools

import jax
import jax.numpy as jnp
from jax import lax
from jax.experimental import pallas as pl
from jax.experimental.pallas import tpu as pltpu
from jax.experimental.pallas import tpu_sc as plsc

NUM_CORES = 2
NUM_SUBCORES = 16
NUM_TILES = NUM_CORES * NUM_SUBCORES
LANES = 16
D = 32
CHUNK = 128
SCORE_BLK = 32768


def _scores_body(ut_ref, mt_ref, wu_ref, wm_ref, us_ref, ms_ref):
    us_ref[...] = jnp.sum(ut_ref[...] * wu_ref[...], axis=0)
    ms_ref[...] = jnp.sum(mt_ref[...] * wm_ref[...], axis=0)


@functools.lru_cache(maxsize=None)
def _build_scores(n_rows: int, d: int, blk: int):
    grid = pl.cdiv(n_rows, blk)
    return pl.pallas_call(
        _scores_body,
        grid=(grid,),
        in_specs=[
            pl.BlockSpec((d, blk), lambda j: (0, j)),
            pl.BlockSpec((d, blk), lambda j: (0, j)),
            pl.BlockSpec((d, 1), lambda j: (0, 0)),
            pl.BlockSpec((d, 1), lambda j: (0, 0)),
        ],
        out_specs=[
            pl.BlockSpec((blk,), lambda j: (j,)),
            pl.BlockSpec((blk,), lambda j: (j,)),
        ],
        out_shape=[jax.ShapeDtypeStruct((n_rows,), jnp.float32)] * 2,
        compiler_params=pltpu.CompilerParams(
            dimension_semantics=("parallel",)),
    )


@functools.lru_cache(maxsize=None)
def _build_gather(batch: int):
    assert batch % (8 * NUM_TILES) == 0
    bpw = batch // NUM_TILES
    n_chunks = bpw // CHUNK

    mesh = plsc.VectorSubcoreMesh(core_axis_name="c", subcore_axis_name="s")
    cp = pltpu.CompilerParams()
    if "needs_layout_passes" in pltpu.CompilerParams.__dataclass_fields__:
        cp = dataclasses.replace(cp, needs_layout_passes=False)

    @functools.partial(
        pl.kernel,
        out_type=jax.ShapeDtypeStruct((batch,), jnp.float32),
        mesh=mesh,
        compiler_params=cp,
        scratch_types=[
            pltpu.VMEM((bpw,), jnp.int32),
            pltpu.VMEM((bpw,), jnp.int32),
            pltpu.VMEM((bpw,), jnp.float32),
            pltpu.VMEM((bpw,), jnp.float32),
            pltpu.VMEM((bpw,), jnp.float32),
            pltpu.VMEM((LANES,), jnp.float32),
            pltpu.SemaphoreType.DMA,
            pltpu.SemaphoreType.DMA,
        ],
    )
    def gather_add(users_h, movies_h, us_h, ms_h, bv_h, out_h,
                   uidx, midx, usv, msv, outv, bvv, sem_u, sem_m):
        wid = lax.axis_index("s") * NUM_CORES + lax.axis_index("c")
        base = wid * bpw

        pltpu.sync_copy(users_h.at[pl.ds(base, bpw)], uidx)
        pltpu.sync_copy(movies_h.at[pl.ds(base, bpw)], midx)
        pltpu.sync_copy(bv_h, bvv)

        copies = []
        for c in range(n_chunks):
            sl = pl.ds(c * CHUNK, CHUNK)
            copies.append(
                pltpu.async_copy(us_h.at[uidx.at[sl]], usv.at[sl], sem_u))
            copies.append(
                pltpu.async_copy(ms_h.at[midx.at[sl]], msv.at[sl], sem_m))
        for cp_ in copies:
            cp_.wait()

        bvec = bvv[...]

        @pl.loop(0, bpw, step=LANES)
        def _(i):
            sl = pl.ds(i, LANES)
            outv[sl] = usv[sl] + msv[sl] + bvec

        pltpu.sync_copy(outv, out_h.at[pl.ds(base, bpw)])

    return gather_add


def kernel(users, movies, user_table, movie_table, W, b):
    batch = users.shape[0]
    n_rows, d = user_table.shape
    users = users.astype(jnp.int32)
    movies = movies.astype(jnp.int32)
    utT = user_table.T
    mtT = movie_table.T
    wu = W[0, :d].reshape(d, 1).astype(jnp.float32)
    wm = W[0, d:].reshape(d, 1).astype(jnp.float32)
    uscore, mscore = _build_scores(n_rows, d, SCORE_BLK)(utT, mtT, wu, wm)
    bv = jnp.broadcast_to(b, (LANES,)).astype(jnp.float32)
    out = _build_gather(batch)(users, movies, uscore, mscore, bv)
    return out.reshape(batch, 1)
